# baseline (speedup 1.0000x reference)
import functools

import jax
import jax.numpy as jnp
from jax import lax
from jax.experimental import pallas as pl
from jax.experimental.pallas import tpu as pltpu
from jax.experimental.pallas import tpu_sc as plsc

_INFO = plsc.get_sparse_core_info()
_NC = _INFO.num_cores
_NS = _INFO.num_subcores
_NW = _NC * _NS

_CHUNK = 512
_BLK = _CHUNK // 128
_NB = 4


@functools.partial(jax.jit, static_argnums=(2, 3, 4))
def _sc_gather(x, table, bsz, h, d):
    nblk = bsz // 128
    dg = d // 8
    ngrp = (h - 8) // 16
    assert bsz == _CHUNK * _NW and d % 8 == 0
    assert h % 16 == 8 and ngrp >= 1
    mesh = plsc.VectorSubcoreMesh(core_axis_name="c", subcore_axis_name="s")

    @functools.partial(
        pl.kernel,
        mesh=mesh,
        out_type=jax.ShapeDtypeStruct((h, dg, nblk, 1024), jnp.float32),
        scratch_types=(
            [pltpu.VMEM((8 * _CHUNK,), jnp.int32) for _ in range(2)]
            + [pltpu.VMEM((_CHUNK, d), jnp.float32) for _ in range(_NB)]
            + [pltpu.VMEM((dg * _BLK, 1024), jnp.float32) for _ in range(2)]
            + [pltpu.VMEM((_CHUNK, 8), jnp.int32) for _ in range(2)]
            + [pltpu.SemaphoreType.DMA for _ in range(_NB + 4)]
        ),
        compiler_params=pltpu.CompilerParams(use_tc_tiling_on_sc=False,
                                             needs_layout_passes=False),
    )
    def k(x_hbm, tab_hbm, out_hbm, *scratch):
        idxT = scratch[0:2]
        rows_v = scratch[2:2 + _NB]
        t_v = scratch[2 + _NB:4 + _NB]
        xb_v = scratch[4 + _NB:6 + _NB]
        sxb = scratch[6 + _NB:8 + _NB]
        sg = scratch[8 + _NB:8 + 2 * _NB]
        so = scratch[8 + 2 * _NB:10 + 2 * _NB]

        wid = lax.axis_index("s") * _NC + lax.axis_index("c")
        b0 = wid * _CHUNK
        lane = lax.iota(jnp.int32, 16)

        def xb_copy(col0, xb):
            return pltpu.make_async_copy(
                x_hbm.at[pl.ds(b0, _CHUNK), pl.ds(col0, 8)],
                xb_v[xb], sxb[xb])

        def gather_copy(koff, tpar, p):
            return pltpu.make_async_copy(
                tab_hbm.at[idxT[tpar].at[pl.ds(koff * _CHUNK, _CHUNK)]],
                rows_v[p], sg[p])

        def out_copy(hh, q, g):
            return pltpu.make_async_copy(
                t_v[q].at[pl.ds(g * _BLK, _BLK)],
                out_hbm.at[hh, g, pl.ds(wid * _BLK, _BLK)], so[q])

        def transpose_x(xb, tpar):
            src = xb_v[xb]
            dst = idxT[tpar]

            def tbody(rg, carry):
                r16 = lane + rg * 16
                for c in range(8):
                    cvec = jnp.full((16,), c, jnp.int32)
                    v = plsc.load_gather(src, [r16, cvec])
                    plsc.store_scatter(dst, [r16 + c * _CHUNK], v)
                return carry

            lax.fori_loop(0, _CHUNK // 16, tbody, 0, unroll=2)

        def transpose_rows(p, q):
            rows = rows_v[p]
            dst = t_v[q]
            colb = (lane % 8) * 128
            for bb in range(_BLK):
                rvecs = [2 * cg * _BLK + bb + (lane // 8) * _BLK
                         for cg in range(d // 16)]

                def jbody(j, carry):
                    row = bb * 128 + j
                    for cg in range(d // 16):
                        v = rows[row, pl.ds(cg * 16, 16)]
                        plsc.store_scatter(dst, [rvecs[cg], colb + j], v)
                    return carry

                lax.fori_loop(0, 128, jbody, 0, unroll=4)

        def body(i, t, start_g):
            p = t % _NB
            q = t % 2
            gather_copy(0, 0, p).wait()
            if start_g:
                gather_copy((t + 3) % 8, ((t + 3) // 8) % 2,
                            (p + 3) % _NB).start()
            for g in range(dg):
                out_copy(0, q, g).wait()
            transpose_rows(p, q)
            for g in range(dg):
                out_copy(i, q, g).start()

        xb_copy(0, 0).start()
        xb_copy(8, 1).start()
        xb_copy(0, 0).wait()
        transpose_x(0, 0)
        for c in range(3):
            gather_copy(c, 0, c).start()
        for q in range(2):
            for g in range(dg):
                out_copy(q, q, g).start()

        def group(m, carry):
            base = 16 * m
            for t in range(16):
                i = base + t
                if t == 5 or t == 13:
                    par = 1 if t == 5 else 0
                    xb_copy(0, par).wait()
                    transpose_x(par, par)
                    xb_copy(jnp.minimum(i + 11, h - 8), 1 - par).start()
                body(i, t, start_g=True)
            return carry

        lax.fori_loop(0, ngrp, group, 0)

        xb_copy(0, 1).wait()
        for t in range(8):
            body(h - 8 + t, t, start_g=(t < 5))

        for q in range(2):
            for g in range(dg):
                out_copy(0, q, g).wait()

    return k(x, table)


def kernel(x, table):
    b, h = x.shape
    v, d = table.shape
    out5 = _sc_gather(x, table, b, h, d)
    out5 = out5.reshape(h, d // 8, b // 128, 8, 128)
    return out5.transpose(2, 4, 0, 1, 3).reshape(b, h, d)
